# baseline (speedup 1.0000x reference)
import functools

import jax
import jax.numpy as jnp
from jax import lax
from jax.experimental import pallas as pl
from jax.experimental.pallas import tpu as pltpu
from jax.experimental.pallas import tpu_sc as plsc

_DIM = 64
_NW = 32
_SUB = 128
_CHUNK = 512
_NSUB = _CHUNK // _SUB


def _emb_body(idx_hbm, table_hbm, out_hbm, idx_v, rows_v, sem):
    b_total = out_hbm.shape[0]
    b_per_w = b_total // _NW
    n_chunks = b_per_w // _CHUNK
    wid = lax.axis_index("s") * 2 + lax.axis_index("c")
    irow0 = wid * (b_per_w // _SUB)
    orow0 = wid * b_per_w

    def chunk(g, carry):
        pltpu.sync_copy(idx_hbm.at[pl.ds(irow0 + g * _NSUB, _NSUB)], idx_v)
        copies = [
            pltpu.async_copy(table_hbm.at[idx_v.at[j]],
                             rows_v.at[pl.ds(j * _SUB, _SUB)], sem)
            for j in range(_NSUB)
        ]
        for c in copies:
            c.wait()
        pltpu.sync_copy(rows_v, out_hbm.at[pl.ds(orow0 + g * _CHUNK, _CHUNK)])
        return carry

    lax.fori_loop(0, n_chunks, chunk, 0)


def kernel(token_ids, weight):
    batch, hist = token_ids.shape
    b_total = batch * hist
    idx2d = token_ids.reshape(b_total // _SUB, _SUB)

    mesh = plsc.VectorSubcoreMesh(core_axis_name="c", subcore_axis_name="s")
    emb = functools.partial(
        pl.kernel,
        mesh=mesh,
        out_type=jax.ShapeDtypeStruct((b_total, _DIM), jnp.float32),
        scratch_types=[
            pltpu.VMEM((_NSUB, _SUB), jnp.int32),
            pltpu.VMEM((_CHUNK, _DIM), jnp.float32),
            pltpu.SemaphoreType.DMA,
        ],
        compiler_params=pltpu.CompilerParams(use_tc_tiling_on_sc=False),
    )(_emb_body)
    out = emb(idx2d, weight)
    return out.reshape(batch, hist, _DIM)


# speedup vs baseline: 1.0753x; 1.0753x over previous
import functools

import jax
import jax.numpy as jnp
from jax import lax
from jax.experimental import pallas as pl
from jax.experimental.pallas import tpu as pltpu
from jax.experimental.pallas import tpu_sc as plsc

_DIM = 64
_NW = 32
_SUB = 128
_CHUNK = 256
_NSUB = _CHUNK // _SUB
_NSLOT = 4


def _emb_body(idx_hbm, table_hbm, out_hbm, idx_v, rows_v, isem, gsem, osem):
    b_total = out_hbm.shape[0]
    b_per_w = b_total // _NW
    n_chunks = b_per_w // _CHUNK
    wid = lax.axis_index("s") * 2 + lax.axis_index("c")
    irow0 = wid * (b_per_w // _SUB)
    orow0 = wid * b_per_w

    def i_copy(g, s):
        return pltpu.make_async_copy(
            idx_hbm.at[pl.ds(irow0 + g * _NSUB, _NSUB)],
            idx_v.at[s], isem.at[s])

    def g_copy(g, s, j):
        return pltpu.make_async_copy(
            table_hbm.at[idx_v.at[s].at[j]],
            rows_v.at[s].at[pl.ds(j * _SUB, _SUB)],
            gsem.at[s])

    def o_copy(g, s):
        return pltpu.make_async_copy(
            rows_v.at[s],
            out_hbm.at[pl.ds(orow0 + g * _CHUNK, _CHUNK)],
            osem.at[s])

    def chunk(g, s, *, wait_o, do_prev, issue_ahead):
        i_copy(g, s).wait()
        if wait_o:
            o_copy(g - _NSLOT, s).wait()
        for j in range(_NSUB):
            g_copy(g, s, j).start()
        if do_prev:
            p = (s - 1) % _NSLOT
            for j in range(_NSUB):
                g_copy(g - 1, p, j).wait()
            o_copy(g - 1, p).start()
            if issue_ahead:
                i_copy(g + _NSLOT - 1, p).start()

    for s in range(_NSLOT):
        i_copy(s, s).start()
    chunk(0, 0, wait_o=False, do_prev=False, issue_ahead=False)
    chunk(1, 1, wait_o=False, do_prev=True, issue_ahead=True)
    chunk(2, 2, wait_o=False, do_prev=True, issue_ahead=True)
    chunk(3, 3, wait_o=False, do_prev=True, issue_ahead=True)

    def body(G, carry):
        g0 = G * _NSLOT
        for s in range(_NSLOT):
            chunk(g0 + s, s, wait_o=True, do_prev=True, issue_ahead=True)
        return carry

    lax.fori_loop(1, n_chunks // _NSLOT - 1, body, 0)

    g0 = n_chunks - _NSLOT
    chunk(g0 + 0, 0, wait_o=True, do_prev=True, issue_ahead=True)
    chunk(g0 + 1, 1, wait_o=True, do_prev=True, issue_ahead=False)
    chunk(g0 + 2, 2, wait_o=True, do_prev=True, issue_ahead=False)
    chunk(g0 + 3, 3, wait_o=True, do_prev=True, issue_ahead=False)

    for j in range(_NSUB):
        g_copy(n_chunks - 1, _NSLOT - 1, j).wait()
    o_copy(n_chunks - 1, _NSLOT - 1).start()
    for s in range(_NSLOT):
        o_copy(n_chunks - _NSLOT + s, s).wait()


def kernel(token_ids, weight):
    batch, hist = token_ids.shape
    b_total = batch * hist
    idx2d = token_ids.reshape(b_total // _SUB, _SUB)

    mesh = plsc.VectorSubcoreMesh(core_axis_name="c", subcore_axis_name="s")
    emb = functools.partial(
        pl.kernel,
        mesh=mesh,
        out_type=jax.ShapeDtypeStruct((b_total, _DIM), jnp.float32),
        scratch_types=[
            pltpu.VMEM((_NSLOT, _NSUB, _SUB), jnp.int32),
            pltpu.VMEM((_NSLOT, _CHUNK, _DIM), jnp.float32),
            pltpu.SemaphoreType.DMA((_NSLOT,)),
            pltpu.SemaphoreType.DMA((_NSLOT,)),
            pltpu.SemaphoreType.DMA((_NSLOT,)),
        ],
        compiler_params=pltpu.CompilerParams(use_tc_tiling_on_sc=False),
    )(_emb_body)
    out = emb(idx2d, weight)
    return out.reshape(batch, hist, _DIM)
